# baseline (speedup 1.0000x reference)
import functools

import jax
import jax.numpy as jnp
from jax import lax
from jax.experimental import pallas as pl
from jax.experimental.pallas import tpu as pltpu
from jax.experimental.pallas import tpu_sc as plsc

NC = 2
NS = 16
NW = NC * NS

SLAB = 128
CHUNK_E = 576
ID_CHUNK = 512


def _proj_body(x_ref, wn_ref, wi_ref, ws_ref, y_ref, z_ref, h_ref):
    xb = x_ref[...]
    y_ref[...] = jnp.dot(xb, wn_ref[...], preferred_element_type=jnp.float32)
    z_ref[...] = jnp.dot(xb, wi_ref[...], preferred_element_type=jnp.float32)
    h_ref[...] = jnp.dot(xb, ws_ref[...], preferred_element_type=jnp.float32)


def _combine_body(h_ref, nb_ref, cnt_ref, idp_ref, bias_ref, out_ref):
    ku = h_ref.shape[1]
    blk = h_ref.shape[0]
    left = h_ref[...] + idp_ref[0] + idp_ref[1]
    cnt = jnp.sum(cnt_ref[...], axis=1)
    right = (nb_ref[0] + nb_ref[1]) / jnp.maximum(cnt[:, None], 1.0)
    bias = bias_ref[...]
    out_ref[:, 0:ku] = jax.nn.relu(left + bias[0, 0:ku])
    out_ref[:, ku:] = jax.nn.relu(right + bias[0, ku:])


def kernel(x, edge_index, id_index, W_self, W_id, W_neighbor, bias):
    n, d = x.shape
    ku = W_self.shape[1]
    e = edge_index.shape[1]
    nid = id_index.shape[0]

    tile_e = e // NW
    full_chunks = tile_e // CHUNK_E
    tail_e = tile_e - full_chunks * CHUNK_E
    n_bodies = full_chunks + (1 if tail_e else 0)
    npad = n + 8
    id_pad = -(-nid // ID_CHUNK) * ID_CHUNK
    id_chunks = id_pad // ID_CHUNK
    rsub = 8 * ((n // NS) // 8)
    rlast = n - rsub * (NS - 1)

    blk = 1000
    grid1 = n // blk
    y, z, h = pl.pallas_call(
        _proj_body,
        grid=(grid1,),
        in_specs=[
            pl.BlockSpec((blk, d), lambda i: (i, 0)),
            pl.BlockSpec((d, ku), lambda i: (0, 0)),
            pl.BlockSpec((d, ku), lambda i: (0, 0)),
            pl.BlockSpec((d, ku), lambda i: (0, 0)),
        ],
        out_specs=[
            pl.BlockSpec((blk, ku), lambda i: (i, 0)),
            pl.BlockSpec((blk, ku), lambda i: (i, 0)),
            pl.BlockSpec((blk, ku), lambda i: (i, 0)),
        ],
        out_shape=[
            jax.ShapeDtypeStruct((n, ku), jnp.float32),
            jax.ShapeDtypeStruct((n, ku), jnp.float32),
            jax.ShapeDtypeStruct((n, ku), jnp.float32),
        ],
    )(x, W_neighbor, W_id, W_self)

    edges = edge_index.astype(jnp.int32)
    idg = jnp.concatenate(
        [id_index.astype(jnp.int32),
         jnp.zeros((id_pad - nid,), dtype=jnp.int32)])
    idsc = jnp.concatenate(
        [id_index.astype(jnp.int32),
         jnp.full((id_pad - nid,), n, dtype=jnp.int32)])
    zeros64 = jnp.zeros((rlast, ku), jnp.float32)

    mesh = plsc.VectorSubcoreMesh(
        core_axis_name="c", subcore_axis_name="s",
        num_cores=NC, num_subcores=NS)

    @functools.partial(
        pl.kernel,
        out_type=(
            jax.ShapeDtypeStruct((NC, n, ku), jnp.float32),
            jax.ShapeDtypeStruct((NW, n), jnp.float32),
            jax.ShapeDtypeStruct((NC, n, ku), jnp.float32),
        ),
        mesh=mesh,
        compiler_params=pltpu.CompilerParams(
            use_tc_tiling_on_sc=False, needs_layout_passes=False),
        scratch_types=[
            pltpu.VMEM((4, CHUNK_E), jnp.int32),
            pltpu.VMEM((4, CHUNK_E), jnp.int32),
            pltpu.VMEM((CHUNK_E, ku), jnp.float32),
            pltpu.VMEM((CHUNK_E, ku), jnp.float32),
            pltpu.VMEM((n,), jnp.float32),
            pltpu.VMEM_SHARED((npad, ku), jnp.float32),
            pltpu.SemaphoreType.DMA,
            pltpu.SemaphoreType.DMA,
            pltpu.SemaphoreType.DMA,
            pltpu.SemaphoreType.DMA,
        ],
    )
    def sc_scatter(y_hbm, z_hbm, edge_hbm, idg_hbm, idsc_hbm, z64_hbm,
                   nb_out, cnt_out, idp_out,
                   row_r, col_r, rows_v0, rows_v1, hist_v,
                   nb_acc, sem_i, sem_g0, sem_g1, sem_s):
        ci = lax.axis_index("c")
        si = lax.axis_index("s")
        wid = si * NC + ci

        r0 = si * rsub

        def zero_nb_async(nrows):
            return pltpu.async_copy(z64_hbm.at[pl.ds(0, nrows)],
                                    nb_acc.at[pl.ds(r0, nrows)], sem_s)

        def zero_nb(nrows):
            zero_nb_async(nrows).wait()

        @pl.when(si < NS - 1)
        def _():
            zd = zero_nb_async(rsub)
            zv = jnp.zeros((16,), jnp.float32)

            @pl.loop(0, n, step=16)
            def _(i):
                hist_v[pl.ds(i, 16)] = zv

            zd.wait()

        @pl.when(si == NS - 1)
        def _():
            zd = zero_nb_async(rlast)
            zv = jnp.zeros((16,), jnp.float32)

            @pl.loop(0, n, step=16)
            def _(i):
                hist_v[pl.ds(i, 16)] = zv

            zd.wait()

        plsc.subcore_barrier()

        ones_reg = jnp.ones((16,), jnp.float32)
        e0 = wid * tile_e
        rows_b = (rows_v0, rows_v1)
        gsem = (sem_g0, sem_g1)
        sizes = [CHUNK_E] * full_chunks + ([tail_e] if tail_e else [])

        def idx_refs(k):
            ne = sizes[k]
            sl = pl.ds(0, ne)
            return row_r.at[k % 4].at[sl], col_r.at[k % 4].at[sl]

        def fire_idx(k):
            base = e0 + k * CHUNK_E
            ne = sizes[k]
            rv, cv = idx_refs(k)
            pltpu.async_copy(edge_hbm.at[0].at[pl.ds(base, ne)], rv, sem_i)
            pltpu.async_copy(edge_hbm.at[1].at[pl.ds(base, ne)], cv, sem_i)

        def wait_idx(k):
            ne = sizes[k]
            rv, cv = idx_refs(k)
            pltpu.make_async_copy(edge_hbm.at[0].at[pl.ds(e0, ne)], rv,
                                  sem_i).wait()
            pltpu.make_async_copy(edge_hbm.at[1].at[pl.ds(e0, ne)], cv,
                                  sem_i).wait()

        def fire_gather(k):
            ne = sizes[k]
            _, cv = idx_refs(k)
            return pltpu.async_copy(
                y_hbm.at[cv], rows_b[k % 2].at[pl.ds(0, ne)], gsem[k % 2])

        fire_idx(0)
        if n_bodies > 1:
            fire_idx(1)
        wait_idx(0)
        gds = {0: fire_gather(0)}
        sds = {}
        for k in range(n_bodies):
            p = k % 2
            ne = sizes[k]
            rv, _ = idx_refs(k)
            if k - 1 in sds:
                sds[k - 1].wait()
            if k + 1 < n_bodies:
                wait_idx(k + 1)
                gds[k + 1] = fire_gather(k + 1)
            if k + 2 < n_bodies:
                fire_idx(k + 2)
            @pl.loop(0, ne, step=16)
            def _(i, _k=k):
                plsc.addupdate_scatter(
                    hist_v, [row_r.at[_k % 4][pl.ds(i, 16)]], ones_reg)

            gds[k].wait()
            sds[k] = pltpu.async_copy(rows_b[p].at[pl.ds(0, ne)],
                                      nb_acc.at[rv], sem_s, add=True)
        sds[n_bodies - 1].wait()

        plsc.subcore_barrier()

        def writeout(dst, nrows):
            sl = pl.ds(r0, nrows)
            pltpu.sync_copy(nb_acc.at[sl], dst.at[ci].at[sl])

        def stage2(nrows):
            sl = pl.ds(r0, nrows)
            wd = pltpu.async_copy(nb_acc.at[sl], nb_out.at[ci].at[sl],
                                  sem_g0)
            hd = pltpu.async_copy(hist_v, cnt_out.at[wid], sem_g1)
            wd.wait()
            zd = zero_nb_async(nrows)
            hd.wait()
            zd.wait()

        @pl.when(si < NS - 1)
        def _():
            stage2(rsub)

        @pl.when(si == NS - 1)
        def _():
            stage2(rlast)

        plsc.subcore_barrier()

        @pl.when(wid < id_chunks)
        def _():
            base = wid * ID_CHUNK
            idv = row_r.at[0].at[pl.ds(0, ID_CHUNK)]
            isv = row_r.at[1].at[pl.ds(0, ID_CHUNK)]
            gv = rows_v0.at[pl.ds(0, ID_CHUNK)]
            pltpu.sync_copy(idg_hbm.at[pl.ds(base, ID_CHUNK)], idv)
            pltpu.sync_copy(idsc_hbm.at[pl.ds(base, ID_CHUNK)], isv)
            pltpu.async_copy(z_hbm.at[idv], gv, sem_g0).wait()
            pltpu.sync_copy(gv, nb_acc.at[isv], add=True)

        plsc.subcore_barrier()

        @pl.when(si < NS - 1)
        def _():
            writeout(idp_out, rsub)

        @pl.when(si == NS - 1)
        def _():
            writeout(idp_out, rlast)

    nb_p, cnt_p, idp_p = sc_scatter(y, z, edges, idg, idsc, zeros64)

    out = pl.pallas_call(
        _combine_body,
        grid=(grid1,),
        in_specs=[
            pl.BlockSpec((blk, ku), lambda i: (i, 0)),
            pl.BlockSpec((NC, blk, ku), lambda i: (0, i, 0)),
            pl.BlockSpec((blk, NW), lambda i: (i, 0)),
            pl.BlockSpec((NC, blk, ku), lambda i: (0, i, 0)),
            pl.BlockSpec((1, 2 * ku), lambda i: (0, 0)),
        ],
        out_specs=pl.BlockSpec((blk, 2 * ku), lambda i: (i, 0)),
        out_shape=jax.ShapeDtypeStruct((n, 2 * ku), jnp.float32),
    )(h, nb_p, cnt_p.T, idp_p, bias.reshape(1, 2 * ku))
    return out
